# Initial kernel scaffold; baseline (speedup 1.0000x reference)
import jax
import jax.numpy as jnp
from jax.experimental import pallas as pl


def kernel(features, Ws0, bs0, Ws1, bs1, Wp0, bp0, Wp1, bp1, Wv0, bv0, Wv1, bv1):
    raise NotImplementedError("write your pallas kernel here")


# speedup vs baseline: 2.6400x; 2.6400x over previous
import functools

import jax
import jax.numpy as jnp
from jax.experimental import pallas as pl
from jax.experimental.pallas import tpu as pltpu

NR_AGENTS = 16
MAX_NB = 16
NB = MAX_NB - 1
UAV_OBS = 130
FEAT_DIM = UAV_OBS - 2
ME_DIM = 256
ME_DIM_SINGLE = NB * UAV_OBS
VFPI_ADD = UAV_OBS + 2 + 64 * 4
FEAT_ALL = ME_DIM_SINGLE + VFPI_ADD
BATCH = 1024
BB = 256
NBB = BATCH // BB


def _fused(n_ref, scale_ref,
           f_ref,
           w0_ref, b0_ref, w1_ref, b1_ref,
           wp0a_ref, wp0b_ref, bp0_ref, wp1_ref, bp1_ref,
           wv0a_ref, wv0b_ref, bv0_ref, wv1_ref, bv1_ref,
           pi_ref, vf_ref,
           acc_ref):
    a = pl.program_id(0)
    n_i = n_ref[a]

    acc_ref[...] = jnp.zeros_like(acc_ref)
    for j in range(NB):
        @pl.when(j < n_i)
        def _():
            x = f_ref[:, UAV_OBS * j:UAV_OBS * j + FEAT_DIM]
            h = jnp.tanh(
                jnp.dot(x, w0_ref[...], preferred_element_type=jnp.float32)
                + b0_ref[...])
            s = jnp.tanh(
                jnp.dot(h, w1_ref[...], preferred_element_type=jnp.float32)
                + b1_ref[...])
            acc_ref[...] += s

    lat = acc_ref[...] * scale_ref[a]
    selfx = f_ref[:, ME_DIM_SINGLE:FEAT_ALL]

    tp = jnp.tanh(
        jnp.dot(lat, wp0a_ref[...], preferred_element_type=jnp.float32)
        + jnp.dot(selfx, wp0b_ref[...], preferred_element_type=jnp.float32)
        + bp0_ref[...])
    pi_ref[...] = jnp.tanh(
        jnp.dot(tp, wp1_ref[...], preferred_element_type=jnp.float32)
        + bp1_ref[...])

    tv = jnp.tanh(
        jnp.dot(lat, wv0a_ref[...], preferred_element_type=jnp.float32)
        + jnp.dot(selfx, wv0b_ref[...], preferred_element_type=jnp.float32)
        + bv0_ref[...])
    vf_ref[...] = jnp.tanh(
        jnp.dot(tv, wv1_ref[...], preferred_element_type=jnp.float32)
        + bv1_ref[...])


@jax.jit
def kernel(features, Ws0, bs0, Ws1, bs1, Wp0, bp0, Wp1, bp1, Wv0, bv0, Wv1, bv1):
    head = features[:NR_AGENTS, :ME_DIM_SINGLE].reshape(NR_AGENTS, NB, UAV_OBS)
    n = jnp.floor(jnp.sum(head[:, :, FEAT_DIM], axis=1))
    n_int = n.astype(jnp.int32)
    scale = jnp.where(n < 1.0, 0.0, 1.0 / jnp.maximum(n, 1.0))

    row2 = lambda b: b.reshape(1, -1)
    grid = (NR_AGENTS, NBB)

    const = lambda *shape: pl.BlockSpec(shape, lambda a, bb, *_: (0,) * len(shape))
    out_shape = jax.ShapeDtypeStruct((NR_AGENTS * BATCH, ME_DIM), jnp.float32)
    out_spec = pl.BlockSpec((BB, ME_DIM), lambda a, bb, *_: (a * NBB + bb, 0))

    pi, vf = pl.pallas_call(
        _fused,
        grid_spec=pltpu.PrefetchScalarGridSpec(
            num_scalar_prefetch=2,
            grid=grid,
            in_specs=[
                pl.BlockSpec((BB, FEAT_ALL), lambda a, bb, *_: (a * NBB + bb, 0)),
                const(FEAT_DIM, 256), const(1, 256),
                const(256, 256), const(1, 256),
                const(256, 256), const(VFPI_ADD, 256), const(1, 256),
                const(256, 256), const(1, 256),
                const(256, 256), const(VFPI_ADD, 256), const(1, 256),
                const(256, 256), const(1, 256),
            ],
            out_specs=[out_spec, out_spec],
            scratch_shapes=[pltpu.VMEM((BB, ME_DIM), jnp.float32)],
        ),
        out_shape=[out_shape, out_shape],
        compiler_params=pltpu.CompilerParams(
            dimension_semantics=("arbitrary", "arbitrary"),
        ),
    )(n_int, scale,
      features,
      Ws0, row2(bs0), Ws1, row2(bs1),
      Wp0[:ME_DIM], Wp0[ME_DIM:], row2(bp0), Wp1, row2(bp1),
      Wv0[:ME_DIM], Wv0[ME_DIM:], row2(bv0), Wv1, row2(bv1))

    pi = pi.reshape(NR_AGENTS, BATCH, ME_DIM)
    vf = vf.reshape(NR_AGENTS, BATCH, ME_DIM)
    return (pi, vf)


# speedup vs baseline: 3.3347x; 1.2632x over previous
import functools

import jax
import jax.numpy as jnp
from jax.experimental import pallas as pl
from jax.experimental.pallas import tpu as pltpu

NR_AGENTS = 16
MAX_NB = 16
NB = MAX_NB - 1
UAV_OBS = 130
FEAT_DIM = UAV_OBS - 2
ME_DIM = 256
ME_DIM_SINGLE = NB * UAV_OBS
VFPI_ADD = UAV_OBS + 2 + 64 * 4
FEAT_ALL = ME_DIM_SINGLE + VFPI_ADD
BATCH = 1024
BB = 256
NBB = BATCH // BB


CHUNK = 3


def _fused(n_ref, w_ref,
           f_ref,
           w0_ref, b0_ref, w1_ref, b1_ref,
           wp0a_ref, wp0b_ref, bp0_ref, wp1_ref, bp1_ref,
           wv0a_ref, wv0b_ref, bv0_ref, wv1_ref, bv1_ref,
           pi_ref, vf_ref,
           acc_ref):
    a = pl.program_id(0)
    n_i = n_ref[a]

    acc_ref[...] = jnp.zeros_like(acc_ref)
    for c in range(0, NB, CHUNK):
        @pl.when(c < n_i)
        def _():
            acc = acc_ref[...]
            for j in range(c, min(c + CHUNK, NB)):
                x = f_ref[:, UAV_OBS * j:UAV_OBS * j + FEAT_DIM]
                h = jnp.tanh(
                    jnp.dot(x, w0_ref[...], preferred_element_type=jnp.float32)
                    + b0_ref[...])
                s = jnp.tanh(
                    jnp.dot(h, w1_ref[...], preferred_element_type=jnp.float32)
                    + b1_ref[...])
                acc = acc + s * w_ref[a, j]
            acc_ref[...] = acc

    lat = acc_ref[...]
    selfx = f_ref[:, ME_DIM_SINGLE:FEAT_ALL]

    tp = jnp.tanh(
        jnp.dot(lat, wp0a_ref[...], preferred_element_type=jnp.float32)
        + jnp.dot(selfx, wp0b_ref[...], preferred_element_type=jnp.float32)
        + bp0_ref[...])
    pi_ref[...] = jnp.tanh(
        jnp.dot(tp, wp1_ref[...], preferred_element_type=jnp.float32)
        + bp1_ref[...])

    tv = jnp.tanh(
        jnp.dot(lat, wv0a_ref[...], preferred_element_type=jnp.float32)
        + jnp.dot(selfx, wv0b_ref[...], preferred_element_type=jnp.float32)
        + bv0_ref[...])
    vf_ref[...] = jnp.tanh(
        jnp.dot(tv, wv1_ref[...], preferred_element_type=jnp.float32)
        + bv1_ref[...])


@jax.jit
def kernel(features, Ws0, bs0, Ws1, bs1, Wp0, bp0, Wp1, bp1, Wv0, bv0, Wv1, bv1):
    head = features[:NR_AGENTS, :ME_DIM_SINGLE].reshape(NR_AGENTS, NB, UAV_OBS)
    n = jnp.floor(jnp.sum(head[:, :, FEAT_DIM], axis=1))
    n_int = n.astype(jnp.int32)
    scale = jnp.where(n < 1.0, 0.0, 1.0 / jnp.maximum(n, 1.0))
    wtab = jnp.where(jnp.arange(NB, dtype=jnp.float32)[None, :] < n[:, None],
                     scale[:, None], 0.0)

    row2 = lambda b: b.reshape(1, -1)
    grid = (NR_AGENTS, NBB)

    const = lambda *shape: pl.BlockSpec(shape, lambda a, bb, *_: (0,) * len(shape))
    out_shape = jax.ShapeDtypeStruct((NR_AGENTS * BATCH, ME_DIM), jnp.float32)
    out_spec = pl.BlockSpec((BB, ME_DIM), lambda a, bb, *_: (a * NBB + bb, 0))

    pi, vf = pl.pallas_call(
        _fused,
        grid_spec=pltpu.PrefetchScalarGridSpec(
            num_scalar_prefetch=2,
            grid=grid,
            in_specs=[
                pl.BlockSpec((BB, FEAT_ALL), lambda a, bb, *_: (a * NBB + bb, 0)),
                const(FEAT_DIM, 256), const(1, 256),
                const(256, 256), const(1, 256),
                const(256, 256), const(VFPI_ADD, 256), const(1, 256),
                const(256, 256), const(1, 256),
                const(256, 256), const(VFPI_ADD, 256), const(1, 256),
                const(256, 256), const(1, 256),
            ],
            out_specs=[out_spec, out_spec],
            scratch_shapes=[pltpu.VMEM((BB, ME_DIM), jnp.float32)],
        ),
        out_shape=[out_shape, out_shape],
        compiler_params=pltpu.CompilerParams(
            dimension_semantics=("arbitrary", "arbitrary"),
        ),
    )(n_int, wtab,
      features,
      Ws0, row2(bs0), Ws1, row2(bs1),
      Wp0[:ME_DIM], Wp0[ME_DIM:], row2(bp0), Wp1, row2(bp1),
      Wv0[:ME_DIM], Wv0[ME_DIM:], row2(bv0), Wv1, row2(bv1))

    pi = pi.reshape(NR_AGENTS, BATCH, ME_DIM)
    vf = vf.reshape(NR_AGENTS, BATCH, ME_DIM)
    return (pi, vf)


# speedup vs baseline: 3.8949x; 1.1680x over previous
import functools

import jax
import jax.numpy as jnp
from jax.experimental import pallas as pl
from jax.experimental.pallas import tpu as pltpu

NR_AGENTS = 16
MAX_NB = 16
NB = MAX_NB - 1
UAV_OBS = 130
FEAT_DIM = UAV_OBS - 2
ME_DIM = 256
ME_DIM_SINGLE = NB * UAV_OBS
VFPI_ADD = UAV_OBS + 2 + 64 * 4
FEAT_ALL = ME_DIM_SINGLE + VFPI_ADD
BATCH = 1024
BB = 512
NBB = BATCH // BB


CHUNK = 3


def _fused(n_ref, w_ref,
           f_ref,
           w0_ref, b0_ref, w1_ref, b1_ref,
           wp0a_ref, wp0b_ref, bp0_ref, wp1_ref, bp1_ref,
           wv0a_ref, wv0b_ref, bv0_ref, wv1_ref, bv1_ref,
           pi_ref, vf_ref,
           acc_ref):
    a = pl.program_id(0)
    n_i = n_ref[a]

    acc_ref[...] = jnp.zeros_like(acc_ref)
    for c in range(0, NB, CHUNK):
        @pl.when(c < n_i)
        def _():
            acc = acc_ref[...]
            for j in range(c, min(c + CHUNK, NB)):
                x = f_ref[:, UAV_OBS * j:UAV_OBS * j + FEAT_DIM]
                h = jnp.tanh(
                    jnp.dot(x, w0_ref[...], preferred_element_type=jnp.float32)
                    + b0_ref[...])
                s = jnp.tanh(
                    jnp.dot(h, w1_ref[...], preferred_element_type=jnp.float32)
                    + b1_ref[...])
                acc = acc + s * w_ref[a, j]
            acc_ref[...] = acc

    lat = acc_ref[...]
    selfx = f_ref[:, ME_DIM_SINGLE:FEAT_ALL]

    tp = jnp.tanh(
        jnp.dot(lat, wp0a_ref[...], preferred_element_type=jnp.float32)
        + jnp.dot(selfx, wp0b_ref[...], preferred_element_type=jnp.float32)
        + bp0_ref[...])
    pi_ref[...] = jnp.tanh(
        jnp.dot(tp, wp1_ref[...], preferred_element_type=jnp.float32)
        + bp1_ref[...])

    tv = jnp.tanh(
        jnp.dot(lat, wv0a_ref[...], preferred_element_type=jnp.float32)
        + jnp.dot(selfx, wv0b_ref[...], preferred_element_type=jnp.float32)
        + bv0_ref[...])
    vf_ref[...] = jnp.tanh(
        jnp.dot(tv, wv1_ref[...], preferred_element_type=jnp.float32)
        + bv1_ref[...])


@jax.jit
def kernel(features, Ws0, bs0, Ws1, bs1, Wp0, bp0, Wp1, bp1, Wv0, bv0, Wv1, bv1):
    head = features[:NR_AGENTS, :ME_DIM_SINGLE].reshape(NR_AGENTS, NB, UAV_OBS)
    n = jnp.floor(jnp.sum(head[:, :, FEAT_DIM], axis=1))
    n_int = n.astype(jnp.int32)
    scale = jnp.where(n < 1.0, 0.0, 1.0 / jnp.maximum(n, 1.0))
    wtab = jnp.where(jnp.arange(NB, dtype=jnp.float32)[None, :] < n[:, None],
                     scale[:, None], 0.0)

    row2 = lambda b: b.reshape(1, -1)
    grid = (NR_AGENTS, NBB)

    const = lambda *shape: pl.BlockSpec(shape, lambda a, bb, *_: (0,) * len(shape))
    out_shape = jax.ShapeDtypeStruct((NR_AGENTS * BATCH, ME_DIM), jnp.float32)
    out_spec = pl.BlockSpec((BB, ME_DIM), lambda a, bb, *_: (a * NBB + bb, 0))

    pi, vf = pl.pallas_call(
        _fused,
        grid_spec=pltpu.PrefetchScalarGridSpec(
            num_scalar_prefetch=2,
            grid=grid,
            in_specs=[
                pl.BlockSpec((BB, FEAT_ALL), lambda a, bb, *_: (a * NBB + bb, 0)),
                const(FEAT_DIM, 256), const(1, 256),
                const(256, 256), const(1, 256),
                const(256, 256), const(VFPI_ADD, 256), const(1, 256),
                const(256, 256), const(1, 256),
                const(256, 256), const(VFPI_ADD, 256), const(1, 256),
                const(256, 256), const(1, 256),
            ],
            out_specs=[out_spec, out_spec],
            scratch_shapes=[pltpu.VMEM((BB, ME_DIM), jnp.float32)],
        ),
        out_shape=[out_shape, out_shape],
        compiler_params=pltpu.CompilerParams(
            dimension_semantics=("arbitrary", "arbitrary"),
        ),
    )(n_int, wtab,
      features,
      Ws0, row2(bs0), Ws1, row2(bs1),
      Wp0[:ME_DIM], Wp0[ME_DIM:], row2(bp0), Wp1, row2(bp1),
      Wv0[:ME_DIM], Wv0[ME_DIM:], row2(bv0), Wv1, row2(bv1))

    pi = pi.reshape(NR_AGENTS, BATCH, ME_DIM)
    vf = vf.reshape(NR_AGENTS, BATCH, ME_DIM)
    return (pi, vf)


# speedup vs baseline: 4.1438x; 1.0639x over previous
import jax
import jax.numpy as jnp
from jax.experimental import pallas as pl
from jax.experimental.pallas import tpu as pltpu

NR_AGENTS = 16
MAX_NB = 16
NB = MAX_NB - 1
UAV_OBS = 130
FEAT_DIM = UAV_OBS - 2
ME_DIM = 256
ME_DIM_SINGLE = NB * UAV_OBS
VFPI_ADD = UAV_OBS + 2 + 64 * 4
FEAT_ALL = ME_DIM_SINGLE + VFPI_ADD
BATCH = 1024
BB = 512
NBB = BATCH // BB
NSTEPS = NR_AGENTS * NBB
CHUNK = 3
NSPLIT = 4
SLAB = BB // NSPLIT


def _fused(n_ref, w_ref,
           f_hbm,
           w0_ref, b0_ref, w1_ref, b1_ref,
           wpv0a_ref, wpv0b_ref, bpv0_ref,
           wp1_ref, bp1_ref, wv1_ref, bv1_ref,
           pi_ref, vf_ref,
           xbuf, acc_ref, sems):
    a = pl.program_id(0)
    bb = pl.program_id(1)
    step = a * NBB + bb
    slot = jax.lax.rem(step, 2)
    n_i = n_ref[a]

    def issue(s, sl):
        for p in range(NSPLIT):
            pltpu.make_async_copy(
                f_hbm.at[pl.ds(s * BB + p * SLAB, SLAB), :],
                xbuf.at[sl, pl.ds(p * SLAB, SLAB)],
                sems.at[sl, p]).start()

    def wait(s, sl):
        for p in range(NSPLIT):
            pltpu.make_async_copy(
                f_hbm.at[pl.ds(s * BB + p * SLAB, SLAB), :],
                xbuf.at[sl, pl.ds(p * SLAB, SLAB)],
                sems.at[sl, p]).wait()

    @pl.when(step == 0)
    def _():
        issue(step, slot)

    @pl.when(step + 1 < NSTEPS)
    def _():
        issue(step + 1, jax.lax.rem(step + 1, 2))

    wait(step, slot)
    f_ref = xbuf.at[slot]

    acc_ref[...] = jnp.zeros_like(acc_ref)
    for c in range(0, NB, CHUNK):
        @pl.when(c < n_i)
        def _():
            acc = acc_ref[...]
            for j in range(c, min(c + CHUNK, NB)):
                x = f_ref[:, UAV_OBS * j:UAV_OBS * j + FEAT_DIM]
                h = jnp.tanh(
                    jnp.dot(x, w0_ref[...], preferred_element_type=jnp.float32)
                    + b0_ref[...])
                s = jnp.tanh(
                    jnp.dot(h, w1_ref[...], preferred_element_type=jnp.float32)
                    + b1_ref[...])
                acc = acc + s * w_ref[a, j]
            acc_ref[...] = acc

    lat = acc_ref[...]
    selfx = f_ref[:, ME_DIM_SINGLE:FEAT_ALL]

    t = jnp.tanh(
        jnp.dot(lat, wpv0a_ref[...], preferred_element_type=jnp.float32)
        + jnp.dot(selfx, wpv0b_ref[...], preferred_element_type=jnp.float32)
        + bpv0_ref[...])
    pi_ref[...] = jnp.tanh(
        jnp.dot(t[:, :ME_DIM], wp1_ref[...],
                preferred_element_type=jnp.float32) + bp1_ref[...])
    vf_ref[...] = jnp.tanh(
        jnp.dot(t[:, ME_DIM:], wv1_ref[...],
                preferred_element_type=jnp.float32) + bv1_ref[...])


@jax.jit
def kernel(features, Ws0, bs0, Ws1, bs1, Wp0, bp0, Wp1, bp1, Wv0, bv0, Wv1, bv1):
    head = features[:NR_AGENTS, :ME_DIM_SINGLE].reshape(NR_AGENTS, NB, UAV_OBS)
    n = jnp.floor(jnp.sum(head[:, :, FEAT_DIM], axis=1))
    n_int = n.astype(jnp.int32)
    scale = jnp.where(n < 1.0, 0.0, 1.0 / jnp.maximum(n, 1.0))
    wtab = jnp.where(jnp.arange(NB, dtype=jnp.float32)[None, :] < n[:, None],
                     scale[:, None], 0.0)

    Wpv0 = jnp.concatenate([Wp0, Wv0], axis=1)
    bpv0 = jnp.concatenate([bp0, bv0])

    row2 = lambda b: b.reshape(1, -1)
    grid = (NR_AGENTS, NBB)

    const = lambda *shape: pl.BlockSpec(shape, lambda a, bb, *_: (0,) * len(shape))
    out_shape = jax.ShapeDtypeStruct((NR_AGENTS * BATCH, ME_DIM), jnp.float32)
    out_spec = pl.BlockSpec((BB, ME_DIM), lambda a, bb, *_: (a * NBB + bb, 0))

    pi, vf = pl.pallas_call(
        _fused,
        grid_spec=pltpu.PrefetchScalarGridSpec(
            num_scalar_prefetch=2,
            grid=grid,
            in_specs=[
                pl.BlockSpec(memory_space=pltpu.MemorySpace.HBM),
                const(FEAT_DIM, 256), const(1, 256),
                const(256, 256), const(1, 256),
                const(ME_DIM, 2 * ME_DIM), const(VFPI_ADD, 2 * ME_DIM),
                const(1, 2 * ME_DIM),
                const(256, 256), const(1, 256),
                const(256, 256), const(1, 256),
            ],
            out_specs=[out_spec, out_spec],
            scratch_shapes=[
                pltpu.VMEM((2, BB, FEAT_ALL), jnp.float32),
                pltpu.VMEM((BB, ME_DIM), jnp.float32),
                pltpu.SemaphoreType.DMA((2, NSPLIT)),
            ],
        ),
        out_shape=[out_shape, out_shape],
        compiler_params=pltpu.CompilerParams(
            dimension_semantics=("arbitrary", "arbitrary"),
        ),
    )(n_int, wtab,
      features,
      Ws0, row2(bs0), Ws1, row2(bs1),
      Wpv0[:ME_DIM], Wpv0[ME_DIM:], row2(bpv0),
      Wp1, row2(bp1), Wv1, row2(bv1))

    pi = pi.reshape(NR_AGENTS, BATCH, ME_DIM)
    vf = vf.reshape(NR_AGENTS, BATCH, ME_DIM)
    return (pi, vf)


# speedup vs baseline: 4.1746x; 1.0074x over previous
import jax
import jax.numpy as jnp
from jax.experimental import pallas as pl
from jax.experimental.pallas import tpu as pltpu

NR_AGENTS = 16
MAX_NB = 16
NB = MAX_NB - 1
UAV_OBS = 130
FEAT_DIM = UAV_OBS - 2
ME_DIM = 256
ME_DIM_SINGLE = NB * UAV_OBS
VFPI_ADD = UAV_OBS + 2 + 64 * 4
FEAT_ALL = ME_DIM_SINGLE + VFPI_ADD
BATCH = 1024
BB = 512
NBB = BATCH // BB
CHUNK = 3


def _fused(n_ref, w_ref,
           f_ref,
           w0_ref, b0_ref, w1_ref, b1_ref,
           wpv0a_ref, wpv0b_ref, bpv0_ref,
           wp1_ref, bp1_ref, wv1_ref, bv1_ref,
           pi_ref, vf_ref,
           acc_ref):
    a = pl.program_id(0)
    n_i = n_ref[a]

    acc_ref[...] = jnp.zeros_like(acc_ref)
    for c in range(0, NB, CHUNK):
        @pl.when(c < n_i)
        def _():
            acc = acc_ref[...]
            for j in range(c, min(c + CHUNK, NB)):
                x = f_ref[:, UAV_OBS * j:UAV_OBS * j + FEAT_DIM]
                h = jnp.tanh(
                    jnp.dot(x, w0_ref[...], preferred_element_type=jnp.float32)
                    + b0_ref[...])
                s = jnp.tanh(
                    jnp.dot(h, w1_ref[...], preferred_element_type=jnp.float32)
                    + b1_ref[...])
                acc = acc + s * w_ref[a, j]
            acc_ref[...] = acc

    lat = acc_ref[...]
    selfx = f_ref[:, ME_DIM_SINGLE:FEAT_ALL]

    t = jnp.tanh(
        jnp.dot(lat, wpv0a_ref[...], preferred_element_type=jnp.float32)
        + jnp.dot(selfx, wpv0b_ref[...], preferred_element_type=jnp.float32)
        + bpv0_ref[...])
    pi_ref[...] = jnp.tanh(
        jnp.dot(t[:, :ME_DIM], wp1_ref[...],
                preferred_element_type=jnp.float32) + bp1_ref[...])
    vf_ref[...] = jnp.tanh(
        jnp.dot(t[:, ME_DIM:], wv1_ref[...],
                preferred_element_type=jnp.float32) + bv1_ref[...])


@jax.jit
def kernel(features, Ws0, bs0, Ws1, bs1, Wp0, bp0, Wp1, bp1, Wv0, bv0, Wv1, bv1):
    head = features[:NR_AGENTS, :ME_DIM_SINGLE].reshape(NR_AGENTS, NB, UAV_OBS)
    n = jnp.floor(jnp.sum(head[:, :, FEAT_DIM], axis=1))
    n_int = n.astype(jnp.int32)
    scale = jnp.where(n < 1.0, 0.0, 1.0 / jnp.maximum(n, 1.0))
    wtab = jnp.where(jnp.arange(NB, dtype=jnp.float32)[None, :] < n[:, None],
                     scale[:, None], 0.0)

    Wpv0 = jnp.concatenate([Wp0, Wv0], axis=1)
    bpv0 = jnp.concatenate([bp0, bv0])

    row2 = lambda b: b.reshape(1, -1)
    grid = (NR_AGENTS, NBB)

    const = lambda *shape: pl.BlockSpec(shape, lambda a, bb, *_: (0,) * len(shape))
    out_shape = jax.ShapeDtypeStruct((NR_AGENTS * BATCH, ME_DIM), jnp.float32)
    out_spec = pl.BlockSpec((BB, ME_DIM), lambda a, bb, *_: (a * NBB + bb, 0))

    pi, vf = pl.pallas_call(
        _fused,
        grid_spec=pltpu.PrefetchScalarGridSpec(
            num_scalar_prefetch=2,
            grid=grid,
            in_specs=[
                pl.BlockSpec((BB, FEAT_ALL), lambda a, bb, *_: (a * NBB + bb, 0)),
                const(FEAT_DIM, 256), const(1, 256),
                const(256, 256), const(1, 256),
                const(ME_DIM, 2 * ME_DIM), const(VFPI_ADD, 2 * ME_DIM),
                const(1, 2 * ME_DIM),
                const(256, 256), const(1, 256),
                const(256, 256), const(1, 256),
            ],
            out_specs=[out_spec, out_spec],
            scratch_shapes=[pltpu.VMEM((BB, ME_DIM), jnp.float32)],
        ),
        out_shape=[out_shape, out_shape],
        compiler_params=pltpu.CompilerParams(
            dimension_semantics=("parallel", "parallel"),
        ),
    )(n_int, wtab,
      features,
      Ws0, row2(bs0), Ws1, row2(bs1),
      Wpv0[:ME_DIM], Wpv0[ME_DIM:], row2(bpv0),
      Wp1, row2(bp1), Wv1, row2(bv1))

    pi = pi.reshape(NR_AGENTS, BATCH, ME_DIM)
    vf = vf.reshape(NR_AGENTS, BATCH, ME_DIM)
    return (pi, vf)
